# Initial kernel scaffold; baseline (speedup 1.0000x reference)
import jax
import jax.numpy as jnp
from jax.experimental import pallas as pl


def kernel(inputs, pos_table):
    raise NotImplementedError("write your pallas kernel here")


# speedup vs baseline: 4.6263x; 4.6263x over previous
import jax
import jax.numpy as jnp
from jax.experimental import pallas as pl

BLOCK = 256


def _bcast_kernel(table_ref, out_ref):
    out_ref[...] = jnp.broadcast_to(table_ref[...][None, :, :], out_ref.shape)


def kernel(inputs, pos_table):
    batch, n_seq = inputs.shape
    d_model = pos_table.shape[1]
    grid = (n_seq // BLOCK,)
    return pl.pallas_call(
        _bcast_kernel,
        grid=grid,
        in_specs=[pl.BlockSpec((BLOCK, d_model), lambda i: (i, 0))],
        out_specs=pl.BlockSpec((batch, BLOCK, d_model), lambda i: (0, i, 0)),
        out_shape=jax.ShapeDtypeStruct((batch, n_seq, d_model), pos_table.dtype),
    )(pos_table)


# speedup vs baseline: 5.0160x; 1.0842x over previous
import jax
import jax.numpy as jnp
from jax.experimental import pallas as pl

BLOCK = 512


def _bcast_kernel(table_ref, out_ref):
    out_ref[...] = jnp.broadcast_to(table_ref[...][None, :, :], out_ref.shape)


def kernel(inputs, pos_table):
    batch, n_seq = inputs.shape
    d_model = pos_table.shape[1]
    grid = (n_seq // BLOCK,)
    return pl.pallas_call(
        _bcast_kernel,
        grid=grid,
        in_specs=[pl.BlockSpec((BLOCK, d_model), lambda i: (i, 0))],
        out_specs=pl.BlockSpec((batch, BLOCK, d_model), lambda i: (0, i, 0)),
        out_shape=jax.ShapeDtypeStruct((batch, n_seq, d_model), pos_table.dtype),
    )(pos_table)
